# Initial kernel scaffold; baseline (speedup 1.0000x reference)
import jax
import jax.numpy as jnp
from jax.experimental import pallas as pl


def kernel(x, edge_index, W_self1, W_neigh1, b1, W_self2, W_neigh2, b2):
    raise NotImplementedError("write your pallas kernel here")


# speedup vs baseline: 13.1595x; 13.1595x over previous
import functools

import jax
import jax.numpy as jnp
from jax import lax
from jax.experimental import pallas as pl
from jax.experimental.pallas import tpu as pltpu
from jax.experimental.pallas import tpu_sc as plsc

N = 10000
E = 320000
F = 128
H = 16

NC = 2
NS = 16
NW = NC * NS
C = 2048
CPW = 5
E_PAD = NW * CPW * C
ROWS_PER_CHUNK = C // 128
N_ACC = 10112
PER_SUB = N_ACC // NS
SINK = N

_mesh = plsc.VectorSubcoreMesh(core_axis_name="c", subcore_axis_name="s")


def _make_sc_agg(with_deg):
    out_types = [jax.ShapeDtypeStruct((NC * N_ACC, H), jnp.float32)]
    scratch = [
        pltpu.VMEM((C,), jnp.int32),
        pltpu.VMEM((C,), jnp.int32),
        pltpu.VMEM((C, H), jnp.float32),
        pltpu.VMEM((PER_SUB, H), jnp.float32),
        pltpu.VMEM_SHARED((N_ACC, H), jnp.float32),
        pltpu.SemaphoreType.DMA,
    ]
    if with_deg:
        out_types.append(jax.ShapeDtypeStruct((NC * N_ACC, H), jnp.float32))
        scratch += [
            pltpu.VMEM((C, H), jnp.float32),
            pltpu.VMEM_SHARED((N_ACC, H), jnp.float32),
        ]

    def body(*refs):
        if with_deg:
            (tab, srci, dsti, out_agg, out_deg,
             srcb, dstb, rows, zb, acc, sem, onesb, dacc) = refs
        else:
            (tab, srci, dsti, out_agg,
             srcb, dstb, rows, zb, acc, sem) = refs
        cid = lax.axis_index("c")
        sid = lax.axis_index("s")
        wid = sid * NC + cid

        def zfill(i, _):
            zb[i, :] = jnp.zeros((H,), jnp.float32)
            return 0
        lax.fori_loop(0, PER_SUB, zfill, 0)
        pltpu.sync_copy(zb, acc.at[pl.ds(sid * PER_SUB, PER_SUB)])
        if with_deg:
            pltpu.sync_copy(zb, dacc.at[pl.ds(sid * PER_SUB, PER_SUB)])

            def ofill(i, _):
                onesb[i, :] = jnp.ones((H,), jnp.float32)
                return 0
            lax.fori_loop(0, C, ofill, 0)
        plsc.subcore_barrier()

        for k in range(CPW):
            e0 = (wid * CPW + k) * C
            pltpu.sync_copy(srci.at[pl.ds(e0, C)], srcb)
            pltpu.async_copy(tab.at[srcb], rows, sem).wait()
            pltpu.sync_copy(dsti.at[pl.ds(e0, C)], dstb)
            pltpu.sync_copy(rows, acc.at[dstb], add=True)
            if with_deg:
                pltpu.sync_copy(onesb, dacc.at[dstb], add=True)

        plsc.subcore_barrier()
        off = cid * N_ACC + sid * PER_SUB
        pltpu.sync_copy(acc.at[pl.ds(sid * PER_SUB, PER_SUB)],
                        out_agg.at[pl.ds(off, PER_SUB)])
        if with_deg:
            pltpu.sync_copy(dacc.at[pl.ds(sid * PER_SUB, PER_SUB)],
                            out_deg.at[pl.ds(off, PER_SUB)])

    return pl.kernel(body, out_type=out_types, mesh=_mesh,
                     scratch_types=scratch,
                     compiler_params=pltpu.CompilerParams(
                         use_tc_tiling_on_sc=False))


_sc_agg_deg = _make_sc_agg(True)
_sc_agg = _make_sc_agg(False)


def _tc_mm1(x, wn, ws):
    def body(x_ref, wn_ref, ws_ref, y_ref, s_ref):
        xb = x_ref[...]
        y_ref[...] = lax.dot(xb, wn_ref[...],
                             preferred_element_type=jnp.float32)
        s_ref[...] = lax.dot(xb, ws_ref[...],
                             preferred_element_type=jnp.float32)

    return pl.pallas_call(
        body,
        grid=(5,),
        in_specs=[
            pl.BlockSpec((2000, F), lambda i: (i, 0)),
            pl.BlockSpec((F, H), lambda i: (0, 0)),
            pl.BlockSpec((F, H), lambda i: (0, 0)),
        ],
        out_specs=[
            pl.BlockSpec((2000, H), lambda i: (i, 0)),
            pl.BlockSpec((2000, H), lambda i: (i, 0)),
        ],
        out_shape=[jax.ShapeDtypeStruct((N, H), jnp.float32)] * 2,
    )(x, wn, ws)


def _tc_layer1(s1, p0, p1, d0, d1, b1):
    def body(s_ref, p0_ref, p1_ref, d0_ref, d1_ref, b_ref, h_ref, r_ref):
        r = 1.0 / jnp.maximum(d0_ref[...] + d1_ref[...], 1.0)
        h = s_ref[...] + (p0_ref[...] + p1_ref[...]) * r + b_ref[...]
        h_ref[...] = jnp.maximum(h, 0.0)
        r_ref[...] = r

    spec = pl.BlockSpec((2000, H), lambda i: (i, 0))
    return pl.pallas_call(
        body,
        grid=(5,),
        in_specs=[spec, spec, spec, spec, spec,
                  pl.BlockSpec((1, H), lambda i: (0, 0))],
        out_specs=[spec, spec],
        out_shape=[jax.ShapeDtypeStruct((N, H), jnp.float32)] * 2,
    )(s1, p0, p1, d0, d1, b1)


def _tc_layer2(h1, q0, q1, rdeg, ws2, wn2, b2):
    ncls = ws2.shape[1]

    def body(h_ref, q0_ref, q1_ref, r_ref, ws_ref, wn_ref, b_ref, o_ref):
        hn = (q0_ref[...] + q1_ref[...]) * r_ref[...]
        o_ref[...] = (
            lax.dot(h_ref[...], ws_ref[...],
                    preferred_element_type=jnp.float32)
            + lax.dot(hn, wn_ref[...], preferred_element_type=jnp.float32)
            + b_ref[...]
        )

    spec = pl.BlockSpec((2000, H), lambda i: (i, 0))
    return pl.pallas_call(
        body,
        grid=(5,),
        in_specs=[spec, spec, spec, spec,
                  pl.BlockSpec((H, ncls), lambda i: (0, 0)),
                  pl.BlockSpec((H, ncls), lambda i: (0, 0)),
                  pl.BlockSpec((1, ncls), lambda i: (0, 0))],
        out_specs=pl.BlockSpec((2000, ncls), lambda i: (i, 0)),
        out_shape=jax.ShapeDtypeStruct((N, ncls), jnp.float32),
    )(h1, q0, q1, rdeg, ws2, wn2, b2)


def kernel(x, edge_index, W_self1, W_neigh1, b1, W_self2, W_neigh2, b2):
    src = edge_index[0]
    dst = edge_index[1]
    pad = E_PAD - E
    srcp = jnp.concatenate([src, jnp.zeros((pad,), jnp.int32)])
    dstp = jnp.concatenate([dst, jnp.full((pad,), SINK, jnp.int32)])

    y1, s1 = _tc_mm1(x, W_neigh1, W_self1)
    aggp, degp = _sc_agg_deg(y1, srcp, dstp)
    h1, rdeg = _tc_layer1(
        s1, aggp[:N], aggp[N_ACC:N_ACC + N],
        degp[:N], degp[N_ACC:N_ACC + N], b1.reshape(1, H))
    (agg2p,) = _sc_agg(h1, srcp, dstp)
    out = _tc_layer2(
        h1, agg2p[:N], agg2p[N_ACC:N_ACC + N], rdeg,
        W_self2, W_neigh2, b2.reshape(1, -1))
    return out


# speedup vs baseline: 18.0878x; 1.3745x over previous
import functools

import jax
import jax.numpy as jnp
from jax import lax
from jax.experimental import pallas as pl
from jax.experimental.pallas import tpu as pltpu
from jax.experimental.pallas import tpu_sc as plsc

N = 10000
E = 320000
F = 128
H = 16

NC = 2
NS = 16
NW = NC * NS
C = 1024
CPW = 10
E_PAD = NW * CPW * C
ROWS_PER_CHUNK = C // 128
N_ACC = 10112
PER_SUB = N_ACC // NS
SINK = N

_mesh = plsc.VectorSubcoreMesh(core_axis_name="c", subcore_axis_name="s")


def _make_sc_agg(with_deg):
    out_types = [jax.ShapeDtypeStruct((NC * N_ACC, H), jnp.float32)]
    scratch = [
        pltpu.VMEM((C,), jnp.int32),
        pltpu.VMEM((C,), jnp.int32),
        pltpu.VMEM((C,), jnp.int32),
        pltpu.VMEM((C,), jnp.int32),
        pltpu.VMEM((C, H), jnp.float32),
        pltpu.VMEM((C, H), jnp.float32),
        pltpu.VMEM((PER_SUB, H), jnp.float32),
        pltpu.VMEM_SHARED((N_ACC, H), jnp.float32),
        pltpu.VMEM_SHARED((N_ACC, H), jnp.float32),
        pltpu.SemaphoreType.DMA,
        pltpu.SemaphoreType.DMA,
    ]
    if with_deg:
        out_types.append(jax.ShapeDtypeStruct((NC * N_ACC, H), jnp.float32))
        scratch += [
            pltpu.VMEM((C, H), jnp.float32),
            pltpu.VMEM_SHARED((N_ACC, H), jnp.float32),
        ]

    def body(*refs):
        if with_deg:
            (tab, srci, dsti, out_agg, out_deg,
             srcb0, srcb1, dstb0, dstb1, rows0, rows1, zb, stab, acc,
             sem0, sem1, onesb, dacc) = refs
        else:
            (tab, srci, dsti, out_agg,
             srcb0, srcb1, dstb0, dstb1, rows0, rows1, zb, stab, acc,
             sem0, sem1) = refs
        srcb = [srcb0, srcb1]
        dstb = [dstb0, dstb1]
        rows = [rows0, rows1]
        sem = [sem0, sem1]
        cid = lax.axis_index("c")
        sid = lax.axis_index("s")
        wid = sid * NC + cid
        sub_rows = pl.ds(sid * PER_SUB, PER_SUB)

        pltpu.sync_copy(tab.at[sub_rows], stab.at[sub_rows])

        def zfill(i, _):
            zb[i, :] = jnp.zeros((H,), jnp.float32)
            return 0
        lax.fori_loop(0, PER_SUB, zfill, 0)
        pltpu.sync_copy(zb, acc.at[sub_rows])
        if with_deg:
            pltpu.sync_copy(zb, dacc.at[sub_rows])

            def ofill(i, _):
                onesb[i, :] = jnp.ones((H,), jnp.float32)
                return 0
            lax.fori_loop(0, C, ofill, 0)
        plsc.subcore_barrier()

        def load_idx(k):
            b = k % 2
            e0 = (wid * CPW + k) * C
            pltpu.sync_copy(srci.at[pl.ds(e0, C)], srcb[b])
            g = pltpu.async_copy(stab.at[srcb[b]], rows[b], sem[b])
            pltpu.sync_copy(dsti.at[pl.ds(e0, C)], dstb[b])
            return g

        gat = load_idx(0)
        for k in range(CPW):
            b = k % 2
            nxt = load_idx(k + 1) if k + 1 < CPW else None
            gat.wait()
            pltpu.sync_copy(rows[b], acc.at[dstb[b]], add=True)
            if with_deg:
                pltpu.sync_copy(onesb, dacc.at[dstb[b]], add=True)
            gat = nxt

        plsc.subcore_barrier()
        off = cid * N_ACC + sid * PER_SUB
        pltpu.sync_copy(acc.at[sub_rows], out_agg.at[pl.ds(off, PER_SUB)])
        if with_deg:
            pltpu.sync_copy(dacc.at[sub_rows],
                            out_deg.at[pl.ds(off, PER_SUB)])

    return pl.kernel(body, out_type=out_types, mesh=_mesh,
                     scratch_types=scratch,
                     compiler_params=pltpu.CompilerParams(
                         use_tc_tiling_on_sc=False))


_sc_agg_deg = _make_sc_agg(True)
_sc_agg = _make_sc_agg(False)


def _tc_mm1(x, wn, ws):
    def body(x_ref, wn_ref, ws_ref, y_ref, s_ref):
        xb = x_ref[...]
        y_ref[...] = lax.dot(xb, wn_ref[...],
                             preferred_element_type=jnp.float32)
        s_ref[...] = lax.dot(xb, ws_ref[...],
                             preferred_element_type=jnp.float32)

    return pl.pallas_call(
        body,
        grid=(5,),
        in_specs=[
            pl.BlockSpec((2000, F), lambda i: (i, 0)),
            pl.BlockSpec((F, H), lambda i: (0, 0)),
            pl.BlockSpec((F, H), lambda i: (0, 0)),
        ],
        out_specs=[
            pl.BlockSpec((2000, H), lambda i: (i, 0)),
            pl.BlockSpec((2000, H), lambda i: (i, 0)),
        ],
        out_shape=[jax.ShapeDtypeStruct((N, H), jnp.float32)] * 2,
    )(x, wn, ws)


def _tc_layer1(s1, p0, p1, d0, d1, b1):
    def body(s_ref, p0_ref, p1_ref, d0_ref, d1_ref, b_ref, h_ref, r_ref):
        r = 1.0 / jnp.maximum(d0_ref[...] + d1_ref[...], 1.0)
        h = s_ref[...] + (p0_ref[...] + p1_ref[...]) * r + b_ref[...]
        h_ref[...] = jnp.maximum(h, 0.0)
        r_ref[...] = r

    spec = pl.BlockSpec((2000, H), lambda i: (i, 0))
    return pl.pallas_call(
        body,
        grid=(5,),
        in_specs=[spec, spec, spec, spec, spec,
                  pl.BlockSpec((1, H), lambda i: (0, 0))],
        out_specs=[spec, spec],
        out_shape=[jax.ShapeDtypeStruct((N, H), jnp.float32)] * 2,
    )(s1, p0, p1, d0, d1, b1)


def _tc_layer2(h1, q0, q1, rdeg, ws2, wn2, b2):
    ncls = ws2.shape[1]

    def body(h_ref, q0_ref, q1_ref, r_ref, ws_ref, wn_ref, b_ref, o_ref):
        hn = (q0_ref[...] + q1_ref[...]) * r_ref[...]
        o_ref[...] = (
            lax.dot(h_ref[...], ws_ref[...],
                    preferred_element_type=jnp.float32)
            + lax.dot(hn, wn_ref[...], preferred_element_type=jnp.float32)
            + b_ref[...]
        )

    spec = pl.BlockSpec((2000, H), lambda i: (i, 0))
    return pl.pallas_call(
        body,
        grid=(5,),
        in_specs=[spec, spec, spec, spec,
                  pl.BlockSpec((H, ncls), lambda i: (0, 0)),
                  pl.BlockSpec((H, ncls), lambda i: (0, 0)),
                  pl.BlockSpec((1, ncls), lambda i: (0, 0))],
        out_specs=pl.BlockSpec((2000, ncls), lambda i: (i, 0)),
        out_shape=jax.ShapeDtypeStruct((N, ncls), jnp.float32),
    )(h1, q0, q1, rdeg, ws2, wn2, b2)


def kernel(x, edge_index, W_self1, W_neigh1, b1, W_self2, W_neigh2, b2):
    src = edge_index[0]
    dst = edge_index[1]
    pad = E_PAD - E
    srcp = jnp.concatenate([src, jnp.zeros((pad,), jnp.int32)])
    dstp = jnp.concatenate([dst, jnp.full((pad,), SINK, jnp.int32)])

    y1, s1 = _tc_mm1(x, W_neigh1, W_self1)
    y1p = jnp.pad(y1, ((0, N_ACC - N), (0, 0)))
    aggp, degp = _sc_agg_deg(y1p, srcp, dstp)
    h1, rdeg = _tc_layer1(
        s1, aggp[:N], aggp[N_ACC:N_ACC + N],
        degp[:N], degp[N_ACC:N_ACC + N], b1.reshape(1, H))
    h1p = jnp.pad(h1, ((0, N_ACC - N), (0, 0)))
    (agg2p,) = _sc_agg(h1p, srcp, dstp)
    out = _tc_layer2(
        h1, agg2p[:N], agg2p[N_ACC:N_ACC + N], rdeg,
        W_self2, W_neigh2, b2.reshape(1, -1))
    return out


# speedup vs baseline: 20.1314x; 1.1130x over previous
import functools

import jax
import jax.numpy as jnp
from jax import lax
from jax.experimental import pallas as pl
from jax.experimental.pallas import tpu as pltpu
from jax.experimental.pallas import tpu_sc as plsc

N = 10000
E = 320000
F = 128
H = 16

NC = 2
NS = 16
NW = NC * NS
EPW = E // NW
C = 1024
CPW = -(-EPW // C)
TAIL = EPW - (CPW - 1) * C
N_ACC = 10112
PER_SUB = N_ACC // NS
NBLK = N_ACC // PER_SUB
SINK = N

_mesh = plsc.VectorSubcoreMesh(core_axis_name="c", subcore_axis_name="s")


def _make_sc_agg(with_deg):
    out_types = [jax.ShapeDtypeStruct((NC * N_ACC, H), jnp.float32)]
    scratch = [
        pltpu.VMEM((C,), jnp.int32),
        pltpu.VMEM((C,), jnp.int32),
        pltpu.VMEM((C,), jnp.int32),
        pltpu.VMEM((C,), jnp.int32),
        pltpu.VMEM((C, H), jnp.float32),
        pltpu.VMEM((C, H), jnp.float32),
        pltpu.VMEM((PER_SUB, H), jnp.float32),
        pltpu.VMEM_SHARED((N_ACC, H), jnp.float32),
        pltpu.VMEM_SHARED((N_ACC, H), jnp.float32),
        pltpu.SemaphoreType.DMA,
        pltpu.SemaphoreType.DMA,
    ]
    if with_deg:
        out_types.append(jax.ShapeDtypeStruct((NC * N_ACC, H), jnp.float32))
        scratch += [
            pltpu.VMEM((C, H), jnp.float32),
            pltpu.VMEM_SHARED((N_ACC, H), jnp.float32),
        ]

    def body(*refs):
        if with_deg:
            (tab, ei, out_agg, out_deg,
             srcb0, srcb1, dstb0, dstb1, rows0, rows1, zb, stab, acc,
             sem0, sem1, onesb, dacc) = refs
        else:
            (tab, ei, out_agg,
             srcb0, srcb1, dstb0, dstb1, rows0, rows1, zb, stab, acc,
             sem0, sem1) = refs
        srcb = [srcb0, srcb1]
        dstb = [dstb0, dstb1]
        rows = [rows0, rows1]
        sem = [sem0, sem1]
        cid = lax.axis_index("c")
        sid = lax.axis_index("s")
        wid = sid * NC + cid
        sub_rows = pl.ds(sid * PER_SUB, PER_SUB)

        pltpu.sync_copy(tab.at[sub_rows], stab.at[sub_rows])

        def zfill(i, _):
            zb[i, :] = jnp.zeros((H,), jnp.float32)
            return 0
        lax.fori_loop(0, PER_SUB, zfill, 0)
        pltpu.sync_copy(zb, acc.at[sub_rows])
        if with_deg:
            pltpu.sync_copy(zb, dacc.at[sub_rows])

            def ofill(i, _):
                onesb[i, :] = jnp.ones((H,), jnp.float32)
                return 0
            lax.fori_loop(0, C, ofill, 0)
        plsc.subcore_barrier()

        def load_idx(k):
            b = k % 2
            e0 = wid * EPW + k * C
            if k < CPW - 1:
                pltpu.sync_copy(ei.at[0, pl.ds(e0, C)], srcb[b])
                g = pltpu.async_copy(stab.at[srcb[b]], rows[b], sem[b])
                pltpu.sync_copy(ei.at[1, pl.ds(e0, C)], dstb[b])
            else:
                pltpu.sync_copy(ei.at[0, pl.ds(e0, TAIL)],
                                srcb[b].at[pl.ds(0, TAIL)])
                for t in range((C - TAIL) // 16):
                    srcb[b][pl.ds(TAIL + t * 16, 16)] = jnp.zeros(
                        (16,), jnp.int32)
                g = pltpu.async_copy(stab.at[srcb[b]], rows[b], sem[b])
                pltpu.sync_copy(ei.at[1, pl.ds(e0, TAIL)],
                                dstb[b].at[pl.ds(0, TAIL)])
                for t in range((C - TAIL) // 16):
                    dstb[b][pl.ds(TAIL + t * 16, 16)] = jnp.full(
                        (16,), SINK, jnp.int32)
            return g

        gat = load_idx(0)
        for k in range(CPW):
            b = k % 2
            nxt = load_idx(k + 1) if k + 1 < CPW else None
            gat.wait()
            pltpu.sync_copy(rows[b], acc.at[dstb[b]], add=True)
            if with_deg:
                pltpu.sync_copy(onesb, dacc.at[dstb[b]], add=True)
            gat = nxt

        plsc.subcore_barrier()
        off = cid * N_ACC + sid * PER_SUB
        pltpu.sync_copy(acc.at[sub_rows], out_agg.at[pl.ds(off, PER_SUB)])
        if with_deg:
            pltpu.sync_copy(dacc.at[sub_rows],
                            out_deg.at[pl.ds(off, PER_SUB)])

    return pl.kernel(body, out_type=out_types, mesh=_mesh,
                     scratch_types=scratch,
                     compiler_params=pltpu.CompilerParams(
                         use_tc_tiling_on_sc=False))


_sc_agg_deg = _make_sc_agg(True)
_sc_agg = _make_sc_agg(False)


def _tc_mm1(x, wn, ws):
    def body(x_ref, wn_ref, ws_ref, y_ref, s_ref):
        xb = x_ref[...]
        y_ref[...] = lax.dot(xb, wn_ref[...],
                             preferred_element_type=jnp.float32)
        s_ref[...] = lax.dot(xb, ws_ref[...],
                             preferred_element_type=jnp.float32)

    return pl.pallas_call(
        body,
        grid=(NBLK,),
        in_specs=[
            pl.BlockSpec((PER_SUB, F), lambda i: (i, 0)),
            pl.BlockSpec((F, H), lambda i: (0, 0)),
            pl.BlockSpec((F, H), lambda i: (0, 0)),
        ],
        out_specs=[
            pl.BlockSpec((PER_SUB, H), lambda i: (i, 0)),
            pl.BlockSpec((PER_SUB, H), lambda i: (i, 0)),
        ],
        out_shape=[jax.ShapeDtypeStruct((N_ACC, H), jnp.float32)] * 2,
    )(x, wn, ws)


def _tc_layer1(s1, aggp, degp, b1):
    def body(s_ref, p0_ref, p1_ref, d0_ref, d1_ref, b_ref, h_ref, r_ref):
        r = 1.0 / jnp.maximum(d0_ref[...] + d1_ref[...], 1.0)
        h = s_ref[...] + (p0_ref[...] + p1_ref[...]) * r + b_ref[...]
        h_ref[...] = jnp.maximum(h, 0.0)
        r_ref[...] = r

    spec = pl.BlockSpec((PER_SUB, H), lambda i: (i, 0))
    spec_hi = pl.BlockSpec((PER_SUB, H), lambda i: (i + NBLK, 0))
    return pl.pallas_call(
        body,
        grid=(NBLK,),
        in_specs=[spec, spec, spec_hi, spec, spec_hi,
                  pl.BlockSpec((1, H), lambda i: (0, 0))],
        out_specs=[spec, spec],
        out_shape=[jax.ShapeDtypeStruct((N_ACC, H), jnp.float32)] * 2,
    )(s1, aggp, aggp, degp, degp, b1)


def _tc_layer2(h1, agg2p, rdeg, ws2, wn2, b2):
    ncls = ws2.shape[1]

    def body(h_ref, q0_ref, q1_ref, r_ref, ws_ref, wn_ref, b_ref, o_ref):
        hn = (q0_ref[...] + q1_ref[...]) * r_ref[...]
        o_ref[...] = (
            lax.dot(h_ref[...], ws_ref[...],
                    preferred_element_type=jnp.float32)
            + lax.dot(hn, wn_ref[...], preferred_element_type=jnp.float32)
            + b_ref[...]
        )

    spec = pl.BlockSpec((PER_SUB, H), lambda i: (i, 0))
    spec_hi = pl.BlockSpec((PER_SUB, H), lambda i: (i + NBLK, 0))
    return pl.pallas_call(
        body,
        grid=(NBLK,),
        in_specs=[spec, spec, spec_hi, spec,
                  pl.BlockSpec((H, ncls), lambda i: (0, 0)),
                  pl.BlockSpec((H, ncls), lambda i: (0, 0)),
                  pl.BlockSpec((1, ncls), lambda i: (0, 0))],
        out_specs=pl.BlockSpec((PER_SUB, ncls), lambda i: (i, 0)),
        out_shape=jax.ShapeDtypeStruct((N, ncls), jnp.float32),
    )(h1, agg2p, agg2p, rdeg, ws2, wn2, b2)


def kernel(x, edge_index, W_self1, W_neigh1, b1, W_self2, W_neigh2, b2):
    y1, s1 = _tc_mm1(x, W_neigh1, W_self1)
    aggp, degp = _sc_agg_deg(y1, edge_index)
    h1, rdeg = _tc_layer1(s1, aggp, degp, b1.reshape(1, H))
    (agg2p,) = _sc_agg(h1, edge_index)
    out = _tc_layer2(h1, agg2p, rdeg, W_self2, W_neigh2, b2.reshape(1, -1))
    return out


# speedup vs baseline: 29.4129x; 1.4610x over previous
import functools

import jax
import jax.numpy as jnp
from jax import lax
from jax.experimental import pallas as pl
from jax.experimental.pallas import tpu as pltpu
from jax.experimental.pallas import tpu_sc as plsc

N = 10000
E = 320000
F = 128
H = 16
CLS = 41

NC = 2
NS = 16
NW = NC * NS
EPW = E // NW
C = 1024
CPW = -(-EPW // C)
TAIL = EPW - (CPW - 1) * C
N_ACC = 10112
PER_SUB = N_ACC // NS
SINK = N

PK = 8
PR = N_ACC // PK
PRN = N // PK

_mesh = plsc.VectorSubcoreMesh(core_axis_name="c", subcore_axis_name="s")


def _make_sc_agg(with_deg):
    out_types = [jax.ShapeDtypeStruct((NC * N_ACC, H), jnp.float32)]
    scratch = [
        pltpu.VMEM((C,), jnp.int32),
        pltpu.VMEM((C,), jnp.int32),
        pltpu.VMEM((C,), jnp.int32),
        pltpu.VMEM((C,), jnp.int32),
        pltpu.VMEM((C, H), jnp.float32),
        pltpu.VMEM((C, H), jnp.float32),
        pltpu.VMEM((PER_SUB, H), jnp.float32),
        pltpu.VMEM_SHARED((N_ACC, H), jnp.float32),
        pltpu.VMEM_SHARED((N_ACC, H), jnp.float32),
        pltpu.SemaphoreType.DMA,
        pltpu.SemaphoreType.DMA,
    ]
    if with_deg:
        out_types.append(jax.ShapeDtypeStruct((NC * N_ACC, H), jnp.float32))
        scratch += [
            pltpu.VMEM((C, H), jnp.float32),
            pltpu.VMEM_SHARED((N_ACC, H), jnp.float32),
        ]

    def body(*refs):
        if with_deg:
            (tab, ei, out_agg, out_deg,
             srcb0, srcb1, dstb0, dstb1, rows0, rows1, zb, stab, acc,
             sem0, sem1, onesb, dacc) = refs
        else:
            (tab, ei, out_agg,
             srcb0, srcb1, dstb0, dstb1, rows0, rows1, zb, stab, acc,
             sem0, sem1) = refs
        srcb = [srcb0, srcb1]
        dstb = [dstb0, dstb1]
        rows = [rows0, rows1]
        sem = [sem0, sem1]
        cid = lax.axis_index("c")
        sid = lax.axis_index("s")
        wid = sid * NC + cid
        sub_rows = pl.ds(sid * PER_SUB, PER_SUB)

        pltpu.sync_copy(tab.at[sub_rows], stab.at[sub_rows])

        def zfill(i, _):
            zb[i, :] = jnp.zeros((H,), jnp.float32)
            return 0
        lax.fori_loop(0, PER_SUB, zfill, 0)
        pltpu.sync_copy(zb, acc.at[sub_rows])
        if with_deg:
            pltpu.sync_copy(zb, dacc.at[sub_rows])

            def ofill(i, _):
                onesb[i, :] = jnp.ones((H,), jnp.float32)
                return 0
            lax.fori_loop(0, C, ofill, 0)
        plsc.subcore_barrier()

        def load_idx(k):
            b = k % 2
            e0 = wid * EPW + k * C
            if k < CPW - 1:
                pltpu.sync_copy(ei.at[0, pl.ds(e0, C)], srcb[b])
                g = pltpu.async_copy(stab.at[srcb[b]], rows[b], sem[b])
                pltpu.sync_copy(ei.at[1, pl.ds(e0, C)], dstb[b])
            else:
                pltpu.sync_copy(ei.at[0, pl.ds(e0, TAIL)],
                                srcb[b].at[pl.ds(0, TAIL)])
                for t in range((C - TAIL) // 16):
                    srcb[b][pl.ds(TAIL + t * 16, 16)] = jnp.zeros(
                        (16,), jnp.int32)
                g = pltpu.async_copy(stab.at[srcb[b]], rows[b], sem[b])
                pltpu.sync_copy(ei.at[1, pl.ds(e0, TAIL)],
                                dstb[b].at[pl.ds(0, TAIL)])
                for t in range((C - TAIL) // 16):
                    dstb[b][pl.ds(TAIL + t * 16, 16)] = jnp.full(
                        (16,), SINK, jnp.int32)
            return g

        gat = load_idx(0)
        for k in range(CPW):
            b = k % 2
            nxt = load_idx(k + 1) if k + 1 < CPW else None
            gat.wait()
            pltpu.sync_copy(rows[b], acc.at[dstb[b]], add=True)
            if with_deg:
                pltpu.sync_copy(onesb, dacc.at[dstb[b]], add=True)
            gat = nxt

        plsc.subcore_barrier()
        off = cid * N_ACC + sid * PER_SUB
        pltpu.sync_copy(acc.at[sub_rows], out_agg.at[pl.ds(off, PER_SUB)])
        if with_deg:
            pltpu.sync_copy(dacc.at[sub_rows],
                            out_deg.at[pl.ds(off, PER_SUB)])

    return pl.kernel(body, out_type=out_types, mesh=_mesh,
                     scratch_types=scratch,
                     compiler_params=pltpu.CompilerParams(
                         use_tc_tiling_on_sc=False))


_sc_agg_deg = _make_sc_agg(True)
_sc_agg = _make_sc_agg(False)


def _bdiag(w, blocks, rows, cols):
    t = jnp.tile(w, (blocks, blocks))
    r = lax.broadcasted_iota(jnp.int32, t.shape, 0)
    c = lax.broadcasted_iota(jnp.int32, t.shape, 1)
    return jnp.where((r // rows) == (c // cols), t, 0.0)


def _tc_mm1(xp, wn, ws):
    def body(x_ref, wn_ref, ws_ref, y_ref, s_ref):
        xb = x_ref[...]
        w8n = _bdiag(wn_ref[...], PK, F, H)
        w8s = _bdiag(ws_ref[...], PK, F, H)
        y_ref[...] = lax.dot(xb, w8n, preferred_element_type=jnp.float32)
        s_ref[...] = lax.dot(xb, w8s, preferred_element_type=jnp.float32)

    return pl.pallas_call(
        body,
        grid=(1,),
        in_specs=[
            pl.BlockSpec((PR, PK * F), lambda i: (0, 0)),
            pl.BlockSpec((F, H), lambda i: (0, 0)),
            pl.BlockSpec((F, H), lambda i: (0, 0)),
        ],
        out_specs=[
            pl.BlockSpec((PR, PK * H), lambda i: (0, 0)),
            pl.BlockSpec((PR, PK * H), lambda i: (0, 0)),
        ],
        out_shape=[jax.ShapeDtypeStruct((PR, PK * H), jnp.float32)] * 2,
    )(xp, wn, ws)


def _tc_layer1(s1p, aggp, degp, b1t):
    def body(s_ref, p0_ref, p1_ref, d0_ref, d1_ref, b_ref, h_ref, r_ref):
        r = 1.0 / jnp.maximum(d0_ref[...] + d1_ref[...], 1.0)
        h = s_ref[...] + (p0_ref[...] + p1_ref[...]) * r + b_ref[...]
        h_ref[...] = jnp.maximum(h, 0.0)
        r_ref[...] = r

    spec = pl.BlockSpec((PR, PK * H), lambda i: (0, 0))
    spec_hi = pl.BlockSpec((PR, PK * H), lambda i: (1, 0))
    return pl.pallas_call(
        body,
        grid=(1,),
        in_specs=[spec, spec, spec_hi, spec, spec_hi,
                  pl.BlockSpec((1, PK * H), lambda i: (0, 0))],
        out_specs=[spec, spec],
        out_shape=[jax.ShapeDtypeStruct((PR, PK * H), jnp.float32)] * 2,
    )(s1p, aggp, aggp, degp, degp, b1t)


def _tc_layer2(h1p, agg2p, rdp, ws2, wn2, b2t):
    def body(h_ref, q0_ref, q1_ref, r_ref, ws_ref, wn_ref, b_ref, o_ref):
        w8s = _bdiag(ws_ref[...], PK, H, CLS)
        w8n = _bdiag(wn_ref[...], PK, H, CLS)
        hn = (q0_ref[...] + q1_ref[...]) * r_ref[...]
        o = (lax.dot(h_ref[...], w8s, preferred_element_type=jnp.float32)
             + lax.dot(hn, w8n, preferred_element_type=jnp.float32)
             + b_ref[...])
        o_ref[...] = o[:PRN, :]

    spec = pl.BlockSpec((PR, PK * H), lambda i: (0, 0))
    spec_hi = pl.BlockSpec((PR, PK * H), lambda i: (1, 0))
    return pl.pallas_call(
        body,
        grid=(1,),
        in_specs=[spec, spec, spec_hi, spec,
                  pl.BlockSpec((H, CLS), lambda i: (0, 0)),
                  pl.BlockSpec((H, CLS), lambda i: (0, 0)),
                  pl.BlockSpec((1, PK * CLS), lambda i: (0, 0))],
        out_specs=pl.BlockSpec((PRN, PK * CLS), lambda i: (0, 0)),
        out_shape=jax.ShapeDtypeStruct((PRN, PK * CLS), jnp.float32),
    )(h1p, agg2p, agg2p, rdp, ws2, wn2, b2t)


def kernel(x, edge_index, W_self1, W_neigh1, b1, W_self2, W_neigh2, b2):
    xp = x.reshape(PRN, PK * F)
    y1p, s1p = _tc_mm1(xp, W_neigh1, W_self1)
    aggp, degp = _sc_agg_deg(y1p.reshape(N_ACC, H), edge_index)
    h1p, rdp = _tc_layer1(
        s1p, aggp.reshape(NC * PR, PK * H), degp.reshape(NC * PR, PK * H),
        jnp.tile(b1, PK).reshape(1, PK * H))
    (agg2p,) = _sc_agg(h1p.reshape(N_ACC, H), edge_index)
    outp = _tc_layer2(
        h1p, agg2p.reshape(NC * PR, PK * H), rdp,
        W_self2, W_neigh2, jnp.tile(b2, PK).reshape(1, PK * CLS))
    return outp.reshape(N, CLS)


# speedup vs baseline: 30.5665x; 1.0392x over previous
import functools

import jax
import jax.numpy as jnp
from jax import lax
from jax.experimental import pallas as pl
from jax.experimental.pallas import tpu as pltpu
from jax.experimental.pallas import tpu_sc as plsc

N = 10000
E = 320000
F = 128
H = 16
CLS = 41

NC = 2
NS = 16
NW = NC * NS
EPW = E // NW
N_ACC = 10112
PER_SUB = N_ACC // NS
SINK = N

PK = 8
PR = N_ACC // PK
PRN = N // PK

_mesh = plsc.VectorSubcoreMesh(core_axis_name="c", subcore_axis_name="s")


def _make_sc_agg(with_deg, C):
    CPW = -(-EPW // C)
    TAIL = EPW - (CPW - 1) * C
    out_types = [jax.ShapeDtypeStruct((NC * N_ACC, H), jnp.float32)]
    scratch = [
        pltpu.VMEM((C,), jnp.int32),
        pltpu.VMEM((C,), jnp.int32),
        pltpu.VMEM((C,), jnp.int32),
        pltpu.VMEM((C,), jnp.int32),
        pltpu.VMEM((C, H), jnp.float32),
        pltpu.VMEM((C, H), jnp.float32),
        pltpu.VMEM((PER_SUB, H), jnp.float32),
        pltpu.VMEM_SHARED((N_ACC, H), jnp.float32),
        pltpu.VMEM_SHARED((N_ACC, H), jnp.float32),
        pltpu.SemaphoreType.DMA,
        pltpu.SemaphoreType.DMA,
    ]
    if with_deg:
        out_types.append(jax.ShapeDtypeStruct((NC * N_ACC, H), jnp.float32))
        scratch += [
            pltpu.VMEM((C, H), jnp.float32),
            pltpu.VMEM_SHARED((N_ACC, H), jnp.float32),
            pltpu.SemaphoreType.DMA,
        ]

    def body(*refs):
        if with_deg:
            (tab, ei, out_agg, out_deg,
             srcb0, srcb1, dstb0, dstb1, rows0, rows1, zb, stab, acc,
             sem0, sem1, onesb, dacc, dsem) = refs
        else:
            (tab, ei, out_agg,
             srcb0, srcb1, dstb0, dstb1, rows0, rows1, zb, stab, acc,
             sem0, sem1) = refs
        srcb = [srcb0, srcb1]
        dstb = [dstb0, dstb1]
        rows = [rows0, rows1]
        sem = [sem0, sem1]
        cid = lax.axis_index("c")
        sid = lax.axis_index("s")
        wid = sid * NC + cid
        sub_rows = pl.ds(sid * PER_SUB, PER_SUB)

        pltpu.sync_copy(tab.at[sub_rows], stab.at[sub_rows])

        def zfill(i, _):
            zb[i, :] = jnp.zeros((H,), jnp.float32)
            return 0
        lax.fori_loop(0, PER_SUB, zfill, 0)
        pltpu.sync_copy(zb, acc.at[sub_rows])
        if with_deg:
            pltpu.sync_copy(zb, dacc.at[sub_rows])

            def ofill(i, _):
                onesb[i, :] = jnp.ones((H,), jnp.float32)
                return 0
            lax.fori_loop(0, C, ofill, 0)
        plsc.subcore_barrier()

        def load_idx(k):
            b = k % 2
            e0 = wid * EPW + k * C
            if k < CPW - 1:
                pltpu.sync_copy(ei.at[0, pl.ds(e0, C)], srcb[b])
                g = pltpu.async_copy(stab.at[srcb[b]], rows[b], sem[b])
                pltpu.sync_copy(ei.at[1, pl.ds(e0, C)], dstb[b])
            else:
                pltpu.sync_copy(ei.at[0, pl.ds(e0, TAIL)],
                                srcb[b].at[pl.ds(0, TAIL)])
                for t in range((C - TAIL) // 16):
                    srcb[b][pl.ds(TAIL + t * 16, 16)] = jnp.zeros(
                        (16,), jnp.int32)
                g = pltpu.async_copy(stab.at[srcb[b]], rows[b], sem[b])
                pltpu.sync_copy(ei.at[1, pl.ds(e0, TAIL)],
                                dstb[b].at[pl.ds(0, TAIL)])
                for t in range((C - TAIL) // 16):
                    dstb[b][pl.ds(TAIL + t * 16, 16)] = jnp.full(
                        (16,), SINK, jnp.int32)
            return g

        gat = load_idx(0)
        dscat = None
        for k in range(CPW):
            b = k % 2
            if dscat is not None:
                dscat.wait()
                dscat = None
            nxt = load_idx(k + 1) if k + 1 < CPW else None
            gat.wait()
            if with_deg:
                dscat = pltpu.async_copy(onesb, dacc.at[dstb[b]], dsem,
                                         add=True)
            pltpu.sync_copy(rows[b], acc.at[dstb[b]], add=True)
            gat = nxt
        if dscat is not None:
            dscat.wait()

        plsc.subcore_barrier()
        off = cid * N_ACC + sid * PER_SUB
        pltpu.sync_copy(acc.at[sub_rows], out_agg.at[pl.ds(off, PER_SUB)])
        if with_deg:
            pltpu.sync_copy(dacc.at[sub_rows],
                            out_deg.at[pl.ds(off, PER_SUB)])

    return pl.kernel(body, out_type=out_types, mesh=_mesh,
                     scratch_types=scratch,
                     compiler_params=pltpu.CompilerParams(
                         use_tc_tiling_on_sc=False))


_sc_agg_deg = _make_sc_agg(True, 1024)
_sc_agg = _make_sc_agg(False, 2048)


def _bdiag(w, blocks, rows, cols):
    t = jnp.tile(w, (blocks, blocks))
    r = lax.broadcasted_iota(jnp.int32, t.shape, 0)
    c = lax.broadcasted_iota(jnp.int32, t.shape, 1)
    return jnp.where((r // rows) == (c // cols), t, 0.0)


def _tc_mm1(xp, wn, ws):
    def body(x_ref, wn_ref, ws_ref, y_ref, s_ref):
        xb = x_ref[...]
        w8n = _bdiag(wn_ref[...], PK, F, H)
        w8s = _bdiag(ws_ref[...], PK, F, H)
        y_ref[...] = lax.dot(xb, w8n, preferred_element_type=jnp.float32)
        s_ref[...] = lax.dot(xb, w8s, preferred_element_type=jnp.float32)

    return pl.pallas_call(
        body,
        grid=(1,),
        in_specs=[
            pl.BlockSpec((PR, PK * F), lambda i: (0, 0)),
            pl.BlockSpec((F, H), lambda i: (0, 0)),
            pl.BlockSpec((F, H), lambda i: (0, 0)),
        ],
        out_specs=[
            pl.BlockSpec((PR, PK * H), lambda i: (0, 0)),
            pl.BlockSpec((PR, PK * H), lambda i: (0, 0)),
        ],
        out_shape=[jax.ShapeDtypeStruct((PR, PK * H), jnp.float32)] * 2,
    )(xp, wn, ws)


def _tc_layer1(s1p, aggp, degp, b1t):
    def body(s_ref, p0_ref, p1_ref, d0_ref, d1_ref, b_ref, h_ref, r_ref):
        r = 1.0 / jnp.maximum(d0_ref[...] + d1_ref[...], 1.0)
        h = s_ref[...] + (p0_ref[...] + p1_ref[...]) * r + b_ref[...]
        h_ref[...] = jnp.maximum(h, 0.0)
        r_ref[...] = r

    spec = pl.BlockSpec((PR, PK * H), lambda i: (0, 0))
    spec_hi = pl.BlockSpec((PR, PK * H), lambda i: (1, 0))
    return pl.pallas_call(
        body,
        grid=(1,),
        in_specs=[spec, spec, spec_hi, spec, spec_hi,
                  pl.BlockSpec((1, PK * H), lambda i: (0, 0))],
        out_specs=[spec, spec],
        out_shape=[jax.ShapeDtypeStruct((PR, PK * H), jnp.float32)] * 2,
    )(s1p, aggp, aggp, degp, degp, b1t)


def _tc_layer2(h1p, agg2p, rdp, ws2, wn2, b2t):
    def body(h_ref, q0_ref, q1_ref, r_ref, ws_ref, wn_ref, b_ref, o_ref):
        w8s = _bdiag(ws_ref[...], PK, H, CLS)
        w8n = _bdiag(wn_ref[...], PK, H, CLS)
        hn = (q0_ref[...] + q1_ref[...]) * r_ref[...]
        o = (lax.dot(h_ref[...], w8s, preferred_element_type=jnp.float32)
             + lax.dot(hn, w8n, preferred_element_type=jnp.float32)
             + b_ref[...])
        o_ref[...] = o[:PRN, :]

    spec = pl.BlockSpec((PR, PK * H), lambda i: (0, 0))
    spec_hi = pl.BlockSpec((PR, PK * H), lambda i: (1, 0))
    return pl.pallas_call(
        body,
        grid=(1,),
        in_specs=[spec, spec, spec_hi, spec,
                  pl.BlockSpec((H, CLS), lambda i: (0, 0)),
                  pl.BlockSpec((H, CLS), lambda i: (0, 0)),
                  pl.BlockSpec((1, PK * CLS), lambda i: (0, 0))],
        out_specs=pl.BlockSpec((PRN, PK * CLS), lambda i: (0, 0)),
        out_shape=jax.ShapeDtypeStruct((PRN, PK * CLS), jnp.float32),
    )(h1p, agg2p, agg2p, rdp, ws2, wn2, b2t)


def kernel(x, edge_index, W_self1, W_neigh1, b1, W_self2, W_neigh2, b2):
    xp = x.reshape(PRN, PK * F)
    y1p, s1p = _tc_mm1(xp, W_neigh1, W_self1)
    aggp, degp = _sc_agg_deg(y1p.reshape(N_ACC, H), edge_index)
    h1p, rdp = _tc_layer1(
        s1p, aggp.reshape(NC * PR, PK * H), degp.reshape(NC * PR, PK * H),
        jnp.tile(b1, PK).reshape(1, PK * H))
    (agg2p,) = _sc_agg(h1p.reshape(N_ACC, H), edge_index)
    outp = _tc_layer2(
        h1p, agg2p.reshape(NC * PR, PK * H), rdp,
        W_self2, W_neigh2, jnp.tile(b2, PK).reshape(1, PK * CLS))
    return outp.reshape(N, CLS)
